# baseline (speedup 1.0000x reference)
import jax
import jax.numpy as jnp
from jax.experimental import pallas as pl
from jax.experimental.pallas import tpu as pltpu

_NB = 2048


def _argmin_kernel(qx_ref, qy_ref, q2_ref, pts_ref, idx_out_ref,
                   run_min, run_idx):
    i = pl.program_id(0)
    nb = pts_ref.shape[1]
    px = pts_ref[0:1, :].astype(jnp.bfloat16).astype(jnp.float32)
    py = pts_ref[1:2, :].astype(jnp.bfloat16).astype(jnp.float32)
    p2 = pts_ref[2:3, :]
    qx = qx_ref[...].astype(jnp.bfloat16).astype(jnp.float32)
    qy = qy_ref[...].astype(jnp.bfloat16).astype(jnp.float32)
    q2 = q2_ref[...]
    m1 = qx * px
    m2 = qy * py
    s = m1 + m2
    t = q2 + p2
    d2 = t - 2.0 * s
    d2 = jnp.maximum(d2, 0.0)
    bmin = jnp.min(d2, axis=1, keepdims=True)
    lane = jax.lax.broadcasted_iota(jnp.int32, d2.shape, 1) + i * nb
    cand = jnp.where(d2 == bmin, lane, jnp.int32(2**30))
    bidx = jnp.min(cand, axis=1, keepdims=True)

    @pl.when(i == 0)
    def _():
        run_min[...] = bmin
        run_idx[...] = bidx

    @pl.when(i > 0)
    def _():
        upd = bmin < run_min[...]
        run_min[...] = jnp.where(upd, bmin, run_min[...])
        run_idx[...] = jnp.where(upd, bidx, run_idx[...])

    @pl.when(i == pl.num_programs(0) - 1)
    def _():
        idx_out_ref[...] = run_idx[...]


def _gather_kernel(idx_ref, krow_ref, prow_ref, psf_ref, xy_ref):
    del idx_ref
    row = krow_ref[...]
    s = jnp.sum(row)
    psf_ref[...] = row / (s + 1e-6)
    xy_ref[...] = prow_ref[...]


def kernel(positions, kernels, queries):
    n = positions.shape[0]
    q = queries.shape[0]
    c, h, w = kernels.shape[1:]
    f = c * h * w

    npad = ((n + _NB - 1) // _NB) * _NB
    pad = npad - n
    big = jnp.float32(1.0e6)
    px = jnp.concatenate([positions[:, 0], jnp.full((pad,), big, jnp.float32)])
    py = jnp.concatenate([positions[:, 1], jnp.full((pad,), big, jnp.float32)])
    p2 = px * px + py * py
    pts = jnp.stack([px, py, p2])

    qx = queries[:, 0:1]
    qy = queries[:, 1:2]
    q2 = jnp.sum(queries ** 2, axis=1)[:, None]

    idx = pl.pallas_call(
        _argmin_kernel,
        grid=(npad // _NB,),
        in_specs=[
            pl.BlockSpec((q, 1), lambda i: (0, 0)),
            pl.BlockSpec((q, 1), lambda i: (0, 0)),
            pl.BlockSpec((q, 1), lambda i: (0, 0)),
            pl.BlockSpec((3, _NB), lambda i: (0, i)),
        ],
        out_specs=pl.BlockSpec((q, 1), lambda i: (0, 0)),
        out_shape=jax.ShapeDtypeStruct((q, 1), jnp.int32),
        scratch_shapes=[
            pltpu.VMEM((q, 1), jnp.float32),
            pltpu.VMEM((q, 1), jnp.int32),
        ],
    )(qx, qy, q2, pts)
    idx = idx[:, 0]

    kflat = kernels.reshape(n, 1, f)
    pos3 = positions.reshape(n, 1, 2)
    psf, xy = pl.pallas_call(
        _gather_kernel,
        grid_spec=pltpu.PrefetchScalarGridSpec(
            num_scalar_prefetch=1,
            grid=(q,),
            in_specs=[
                pl.BlockSpec((1, 1, f), lambda i, idx_ref: (idx_ref[i], 0, 0)),
                pl.BlockSpec((1, 1, 2), lambda i, idx_ref: (idx_ref[i], 0, 0)),
            ],
            out_specs=[
                pl.BlockSpec((1, 1, f), lambda i, idx_ref: (i, 0, 0)),
                pl.BlockSpec((1, 1, 2), lambda i, idx_ref: (i, 0, 0)),
            ],
        ),
        out_shape=[
            jax.ShapeDtypeStruct((q, 1, f), jnp.float32),
            jax.ShapeDtypeStruct((q, 1, 2), jnp.float32),
        ],
    )(idx, kflat, pos3)

    x_sel = xy[:, 0, 0]
    y_sel = xy[:, 0, 1]
    psf_sel = psf.reshape(q, c, h, w)
    return (x_sel, y_sel, psf_sel)


# speedup vs baseline: 1.3863x; 1.3863x over previous
import jax
import jax.numpy as jnp
from jax.experimental import pallas as pl
from jax.experimental.pallas import tpu as pltpu

_NB = 2048


def _argmin_kernel(qx_ref, qy_ref, q2_ref, pts_ref, idx_out_ref,
                   run_min, run_idx):
    i = pl.program_id(0)
    nb = pts_ref.shape[1]
    px = pts_ref[0:1, :].astype(jnp.bfloat16).astype(jnp.float32)
    py = pts_ref[1:2, :].astype(jnp.bfloat16).astype(jnp.float32)
    p2 = pts_ref[2:3, :]
    qx = qx_ref[...].astype(jnp.bfloat16).astype(jnp.float32)
    qy = qy_ref[...].astype(jnp.bfloat16).astype(jnp.float32)
    q2 = q2_ref[...]
    m1 = qx * px
    m2 = qy * py
    s = m1 + m2
    t = q2 + p2
    d2 = t - 2.0 * s
    d2 = jnp.maximum(d2, 0.0)
    bmin = jnp.min(d2, axis=1, keepdims=True)
    lane = jax.lax.broadcasted_iota(jnp.int32, d2.shape, 1) + i * nb
    cand = jnp.where(d2 == bmin, lane, jnp.int32(2**30))
    bidx = jnp.min(cand, axis=1, keepdims=True)

    @pl.when(i == 0)
    def _():
        run_min[...] = bmin
        run_idx[...] = bidx

    @pl.when(i > 0)
    def _():
        upd = bmin < run_min[...]
        run_min[...] = jnp.where(upd, bmin, run_min[...])
        run_idx[...] = jnp.where(upd, bidx, run_idx[...])

    @pl.when(i == pl.num_programs(0) - 1)
    def _():
        idx_out_ref[...] = run_idx[...]


_G = 16


def _gather_kernel(idx_ref, *refs):
    del idx_ref
    krows = refs[:_G]
    prows = refs[_G:2 * _G]
    psf_ref = refs[2 * _G]
    xy_ref = refs[2 * _G + 1]
    for j in range(_G):
        row = krows[j][...]
        s = jnp.sum(row)
        psf_ref[j:j + 1, :, :] = row / (s + 1e-6)
        xy_ref[j:j + 1, :, :] = prows[j][...]


def kernel(positions, kernels, queries):
    n = positions.shape[0]
    q = queries.shape[0]
    c, h, w = kernels.shape[1:]
    f = c * h * w

    npad = ((n + _NB - 1) // _NB) * _NB
    pad = npad - n
    big = jnp.float32(1.0e6)
    px = jnp.concatenate([positions[:, 0], jnp.full((pad,), big, jnp.float32)])
    py = jnp.concatenate([positions[:, 1], jnp.full((pad,), big, jnp.float32)])
    p2 = px * px + py * py
    pts = jnp.stack([px, py, p2])

    qx = queries[:, 0:1]
    qy = queries[:, 1:2]
    q2 = jnp.sum(queries ** 2, axis=1)[:, None]

    idx = pl.pallas_call(
        _argmin_kernel,
        grid=(npad // _NB,),
        in_specs=[
            pl.BlockSpec((q, 1), lambda i: (0, 0)),
            pl.BlockSpec((q, 1), lambda i: (0, 0)),
            pl.BlockSpec((q, 1), lambda i: (0, 0)),
            pl.BlockSpec((3, _NB), lambda i: (0, i)),
        ],
        out_specs=pl.BlockSpec((q, 1), lambda i: (0, 0)),
        out_shape=jax.ShapeDtypeStruct((q, 1), jnp.int32),
        scratch_shapes=[
            pltpu.VMEM((q, 1), jnp.float32),
            pltpu.VMEM((q, 1), jnp.int32),
        ],
    )(qx, qy, q2, pts)
    idx = idx[:, 0]

    kflat = kernels.reshape(n, 1, f)
    pos3 = positions.reshape(n, 1, 2)
    k_specs = [
        pl.BlockSpec((1, 1, f), lambda i, idx_ref, j=j: (idx_ref[i * _G + j], 0, 0))
        for j in range(_G)
    ]
    p_specs = [
        pl.BlockSpec((1, 1, 2), lambda i, idx_ref, j=j: (idx_ref[i * _G + j], 0, 0))
        for j in range(_G)
    ]
    psf, xy = pl.pallas_call(
        _gather_kernel,
        grid_spec=pltpu.PrefetchScalarGridSpec(
            num_scalar_prefetch=1,
            grid=(q // _G,),
            in_specs=k_specs + p_specs,
            out_specs=[
                pl.BlockSpec((_G, 1, f), lambda i, idx_ref: (i, 0, 0)),
                pl.BlockSpec((_G, 1, 2), lambda i, idx_ref: (i, 0, 0)),
            ],
        ),
        out_shape=[
            jax.ShapeDtypeStruct((q, 1, f), jnp.float32),
            jax.ShapeDtypeStruct((q, 1, 2), jnp.float32),
        ],
    )(idx, *([kflat] * _G), *([pos3] * _G))

    x_sel = xy[:, 0, 0]
    y_sel = xy[:, 0, 1]
    psf_sel = psf.reshape(q, c, h, w)
    return (x_sel, y_sel, psf_sel)
